# Initial kernel scaffold; baseline (speedup 1.0000x reference)
import jax
import jax.numpy as jnp
from jax.experimental import pallas as pl


def kernel(x, edge_index, edge_attr, batch, W1, b1, W2, b2, W3, b3, W4, b4, g1w, g1b, g1m, g2w, g2b, g2m, g3w, g3b, g3m, g4w, g4b, g4m, p0, p1, p2, p3, l12W, l12b, l3W, l3b):
    raise NotImplementedError("write your pallas kernel here")


# speedup vs baseline: 6.2115x; 6.2115x over previous
import functools

import jax
import jax.numpy as jnp
from jax import lax
from jax.experimental import pallas as pl
from jax.experimental.pallas import tpu as pltpu
from jax.experimental.pallas import tpu_sc as plsc

N = 50000
E = 800000
NG = 64

NPAD = 50176
HALF = NPAD // 2
E_PAD = 802816
EP = E_PAD // 32
EPT = E_PAD // 16
CHK = 3136
NCHK = EP // CHK
NCHK_AGG = EPT // CHK
G = 128
STG = 6400
FLUSH_AT = 3072
GARBAGE = NPAD - 1

_f32 = jnp.float32
_i32 = jnp.int32


def _deg_body(dst_hbm, ew_hbm, out_hbm, dstbuf, ewbuf, hist_a, hist_b, sem):
    core = lax.axis_index("c")
    sub = lax.axis_index("s")
    tid = sub * 2 + core
    base = tid * EP

    def zero_body(i, _):
        z = jnp.zeros((16,), _f32)
        hist_a[pl.ds(i * 16, 16)] = z
        hist_b[pl.ds(i * 16, 16)] = z
        return 0

    lax.fori_loop(0, NPAD // 16, zero_body, 0)

    ones = jnp.ones((16,), _f32)

    def chunk_body(c, _):
        off = base + c * CHK
        pltpu.sync_copy(dst_hbm.at[pl.ds(off, CHK)], dstbuf)
        pltpu.sync_copy(ew_hbm.at[pl.ds(off, CHK)], ewbuf)

        def vec_body(v, _):
            d = dstbuf[pl.ds(v * 16, 16)]
            w = ewbuf[pl.ds(v * 16, 16)]
            plsc.addupdate_scatter(hist_a, [d], ones)
            plsc.addupdate_scatter(hist_b, [d], w)
            return 0

        lax.fori_loop(0, CHK // 16, vec_body, 0)
        return 0

    lax.fori_loop(0, NCHK, chunk_body, 0)

    pltpu.sync_copy(hist_a, out_hbm.at[0, tid])
    pltpu.sync_copy(hist_b, out_hbm.at[1, tid])


def _deg_call(dst_pad, ew_pad):
    mesh = plsc.VectorSubcoreMesh(core_axis_name="c", subcore_axis_name="s")
    return pl.kernel(
        _deg_body,
        out_type=jax.ShapeDtypeStruct((2, 32, NPAD), _f32),
        mesh=mesh,
        scratch_types=[
            pltpu.VMEM((CHK,), _i32),
            pltpu.VMEM((CHK,), _f32),
            pltpu.VMEM((NPAD,), _f32),
            pltpu.VMEM((NPAD,), _f32),
            pltpu.SemaphoreType.DMA,
        ],
        compiler_params=pltpu.CompilerParams(needs_layout_passes=False),
        name="sc_deg",
    )(dst_pad, ew_pad)


def _agg_body(*refs, NT, S, R, use_ew):
    t_hbms = refs[:NT]
    src_hbm, dst_hbm, ew_hbm = refs[NT:NT + 3]
    out_hbms = refs[NT + 3:2 * NT + 3]
    sc = refs[2 * NT + 3:]
    srcbuf, dstbuf, ewbuf, stg_src, stg_dst, stg_ew, idx2d = sc[:7]
    rowbufs = sc[7:7 + NT]
    accs = sc[7 + NT:7 + 2 * NT]
    sem = sc[-1]

    core = lax.axis_index("c")
    sub = lax.axis_index("s")
    base = sub * EPT
    rpt = R // 16
    nvec_row = 128 // 16
    lanes = lax.iota(_i32, 16)

    def pad_stage(cnt):
        q = (cnt // 16) * 16
        m_pad = (q + lanes) >= cnt
        plsc.store_scatter(stg_src, [q + lanes], jnp.zeros((16,), _i32),
                           mask=m_pad)
        plsc.store_scatter(stg_dst, [q + lanes],
                           jnp.full((16,), R, _i32), mask=m_pad)
        if use_ew:
            plsc.store_scatter(stg_ew, [q + lanes], jnp.zeros((16,), _f32),
                               mask=m_pad)
        zi = jnp.zeros((16,), _i32)
        zf = jnp.zeros((16,), _f32)
        gr = jnp.full((16,), R, _i32)
        for j in range(1, 10):
            stg_src[pl.ds(q + 16 * j, 16)] = zi
            stg_dst[pl.ds(q + 16 * j, 16)] = gr
            if use_ew:
                stg_ew[pl.ds(q + 16 * j, 16)] = zf

    def flush(cnt):
        pad_stage(cnt)
        nb = (cnt + (G - 1)) // G

        def batch_body(j, _):
            bb = j * G
            for k in range(G // 16):
                idx2d[0, pl.ds(k * 16, 16)] = stg_dst[pl.ds(bb + k * 16, 16)]
            copies = [
                pltpu.async_copy(t_hbms[n].at[stg_src.at[pl.ds(bb, G)]],
                                 rowbufs[n], sem)
                for n in range(NT)
            ]
            for cp in copies:
                cp.wait()
            if use_ew:
                def scale_body(q, _):
                    wv = stg_ew[pl.ds(bb + q * 16, 16)]
                    for l in range(16):
                        s = wv[l]
                        e = q * 16 + l
                        for n in range(NT):
                            for k in range(nvec_row):
                                rowbufs[n][e, pl.ds(k * 16, 16)] = (
                                    rowbufs[n][e, pl.ds(k * 16, 16)] * s)
                    return 0
                lax.fori_loop(0, G // 16, scale_body, 0)
            for n in range(NT):
                pltpu.sync_copy(rowbufs[n], accs[n].at[idx2d.at[0]],
                                add=True)
            return 0

        lax.fori_loop(0, nb, batch_body, 0)

    def pass_body(p, _):
        lo = core * HALF + p * R

        def zr_body(r, _):
            for n in range(NT):
                for k in range(nvec_row):
                    rowbufs[n][r, pl.ds(k * 16, 16)] = jnp.zeros((16,), _f32)
            return 0
        lax.fori_loop(0, G, zr_body, 0)

        nfull = rpt // G
        rem = rpt % G
        for n in range(NT):
            for j in range(nfull):
                pltpu.sync_copy(rowbufs[n],
                                accs[n].at[pl.ds(sub * rpt + j * G, G)])
            if rem:
                pltpu.sync_copy(rowbufs[n].at[pl.ds(0, rem)],
                                accs[n].at[pl.ds(sub * rpt + nfull * G, rem)])
        plsc.subcore_barrier()

        def chunk_body(c, cnt):
            off = base + c * CHK
            pltpu.sync_copy(src_hbm.at[pl.ds(off, CHK)], srcbuf)
            pltpu.sync_copy(dst_hbm.at[pl.ds(off, CHK)], dstbuf)
            if use_ew:
                pltpu.sync_copy(ew_hbm.at[pl.ds(off, CHK)], ewbuf)

            def vec_body(v, cnt):
                d = dstbuf[pl.ds(v * 16, 16)]
                s = srcbuf[pl.ds(v * 16, 16)]
                m = (d >= lo) & (d < lo + R)
                mi = m.astype(_i32)
                cs = plsc.cumsum(mi)
                pos = cnt + cs - 1
                plsc.store_scatter(stg_src, [pos], s, mask=m)
                plsc.store_scatter(stg_dst, [pos], d - lo, mask=m)
                if use_ew:
                    w = ewbuf[pl.ds(v * 16, 16)]
                    plsc.store_scatter(stg_ew, [pos], w, mask=m)
                return cnt + jnp.sum(mi)

            cnt = lax.fori_loop(0, CHK // 16, vec_body, cnt)

            def do_flush(c2):
                flush(c2)
                return 0

            return lax.cond(cnt >= FLUSH_AT, do_flush, lambda c2: c2, cnt)

        cnt = lax.fori_loop(0, NCHK_AGG, chunk_body, 0)
        lax.cond(cnt > 0, do_flush_final := (lambda c2: (flush(c2), 0)[1]),
                 lambda c2: 0, cnt)
        plsc.subcore_barrier()

        for n in range(NT):
            pltpu.sync_copy(accs[n].at[pl.ds(sub * rpt, rpt)],
                            out_hbms[n].at[pl.ds(lo + sub * rpt, rpt)])
        plsc.subcore_barrier()
        return 0

    lax.fori_loop(0, S, pass_body, 0)


def _agg_call(tables, src_pad, dst_pad, ew_pad, use_ew):
    NT = len(tables)
    S = {1: 4, 2: 7}[NT]
    R = HALF // S
    mesh = plsc.VectorSubcoreMesh(core_axis_name="c", subcore_axis_name="s")
    body = functools.partial(_agg_body, NT=NT, S=S, R=R, use_ew=use_ew)
    out = pl.kernel(
        body,
        out_type=[jax.ShapeDtypeStruct((NPAD, 128), _f32)] * NT,
        mesh=mesh,
        scratch_types=(
            [
                pltpu.VMEM((CHK,), _i32),
                pltpu.VMEM((CHK,), _i32),
                pltpu.VMEM((CHK,), _f32),
                pltpu.VMEM((STG,), _i32),
                pltpu.VMEM((STG,), _i32),
                pltpu.VMEM((STG,), _f32),
                pltpu.VMEM((1, G), _i32),
            ]
            + [pltpu.VMEM((G, 128), _f32)] * NT
            + [pltpu.VMEM_SHARED((R + 16, 128), _f32)] * NT
            + [pltpu.SemaphoreType.DMA]
        ),
        compiler_params=pltpu.CompilerParams(needs_layout_passes=False),
        name=f"sc_agg_nt{NT}",
    )(*tables, src_pad, dst_pad, ew_pad)
    return list(out) if isinstance(out, (list, tuple)) else [out]


BLK = 1024
NBLK = NPAD // BLK
LAST_VALID = N - (NBLK - 1) * BLK


def _row_mask(nrows):
    i = pl.program_id(0)
    rows = jax.lax.broadcasted_iota(_i32, (nrows, 1), 0) + i * BLK
    return rows < N


def _tc_a_body(degp_ref, x_ref, da_ref, db_ref, t1_ref):
    part = degp_ref[...]
    deg_a = jnp.sum(part[:32], axis=0) + 1.0
    deg_b = jnp.sum(part[32:], axis=0) + 1.0
    da = jax.lax.rsqrt(deg_a)[:, None]
    db = jax.lax.rsqrt(deg_b)[:, None]
    da_ref[...] = da
    db_ref[...] = db
    t1 = da * x_ref[...]
    t1_ref[...] = jnp.concatenate(
        [t1, jnp.zeros((t1.shape[0], 96), _f32)], axis=1)


def _tc_a(degpart2, x):
    return pl.pallas_call(
        _tc_a_body,
        grid=(NBLK,),
        in_specs=[
            pl.BlockSpec((64, BLK), lambda i: (0, i)),
            pl.BlockSpec((BLK, 32), lambda i: (i, 0)),
        ],
        out_specs=[
            pl.BlockSpec((BLK, 1), lambda i: (i, 0)),
            pl.BlockSpec((BLK, 1), lambda i: (i, 0)),
            pl.BlockSpec((BLK, 128), lambda i: (i, 0)),
        ],
        out_shape=[
            jax.ShapeDtypeStruct((NPAD, 1), _f32),
            jax.ShapeDtypeStruct((NPAD, 1), _f32),
            jax.ShapeDtypeStruct((N, 128), _f32),
        ],
    )(degpart2, x)


def _stats_accum(i, h, st_ref):
    mask = _row_mask(h.shape[0])
    hm = jnp.where(mask, h, 0.0)
    s1 = jnp.sum(hm, axis=0, keepdims=True)
    s2 = jnp.sum(hm * hm, axis=0, keepdims=True)
    st = jnp.concatenate([s1, s2, jnp.zeros((6, h.shape[1]), _f32)], axis=0)

    @pl.when(i == 0)
    def _():
        st_ref[...] = st

    @pl.when(i > 0)
    def _():
        st_ref[...] = st_ref[...] + st


def _conv_stats_body(*refs, prelu_here, nagg):
    agg_refs = refs[:nagg]
    hprev_ref, dinv_ref, W_ref, b_ref, p_ref, out_ref, st_ref = refs[nagg:]
    i = pl.program_id(0)
    dv = dinv_ref[...]
    agg = jnp.concatenate([r[...] for r in agg_refs], axis=1) \
        if nagg > 1 else agg_refs[0][...]
    u = dv * agg + (dv * dv) * hprev_ref[...]
    h = jnp.dot(u, W_ref[...], preferred_element_type=_f32) + b_ref[0]
    if prelu_here:
        h = jnp.where(h >= 0, h, p_ref[0] * h)
    out_ref[...] = h
    _stats_accum(i, h, st_ref)


def _conv_stats(aggs, hprev, dinv, W, b, p, prelu_here):
    Cin, Cout = W.shape
    Ca = aggs[0].shape[1]
    body = functools.partial(_conv_stats_body, prelu_here=prelu_here,
                             nagg=len(aggs))
    return pl.pallas_call(
        body,
        grid=(NBLK,),
        in_specs=[pl.BlockSpec((BLK, Ca), lambda i: (i, 0))] * len(aggs)
        + [
            pl.BlockSpec((BLK, Cin), lambda i: (i, 0)),
            pl.BlockSpec((BLK, 1), lambda i: (i, 0)),
            pl.BlockSpec((Cin, Cout), lambda i: (0, 0)),
            pl.BlockSpec((1, Cout), lambda i: (0, 0)),
            pl.BlockSpec((1, Cout), lambda i: (0, 0)),
        ],
        out_specs=[
            pl.BlockSpec((BLK, Cout), lambda i: (i, 0)),
            pl.BlockSpec((8, Cout), lambda i: (0, 0)),
        ],
        out_shape=[
            jax.ShapeDtypeStruct((N, Cout), _f32),
            jax.ShapeDtypeStruct((8, Cout), _f32),
        ],
    )(*aggs, hprev, dinv, W, b, p)


def _postagg_stats_body(agg_ref, m_ref, dinv_ref, b_ref, p_ref,
                        out_ref, st_ref):
    i = pl.program_id(0)
    dv = dinv_ref[...]
    h = dv * agg_ref[...] + (dv * dv) * m_ref[...] + b_ref[0]
    h = jnp.where(h >= 0, h, p_ref[0] * h)
    out_ref[...] = h
    _stats_accum(i, h, st_ref)


def _postagg_stats(agg, m, dinv, b, p):
    Cout = m.shape[1]
    return pl.pallas_call(
        _postagg_stats_body,
        grid=(NBLK,),
        in_specs=[
            pl.BlockSpec((BLK, Cout), lambda i: (i, 0)),
            pl.BlockSpec((BLK, Cout), lambda i: (i, 0)),
            pl.BlockSpec((BLK, 1), lambda i: (i, 0)),
            pl.BlockSpec((1, Cout), lambda i: (0, 0)),
            pl.BlockSpec((1, Cout), lambda i: (0, 0)),
        ],
        out_specs=[
            pl.BlockSpec((BLK, Cout), lambda i: (i, 0)),
            pl.BlockSpec((8, Cout), lambda i: (0, 0)),
        ],
        out_shape=[
            jax.ShapeDtypeStruct((N, Cout), _f32),
            jax.ShapeDtypeStruct((8, Cout), _f32),
        ],
    )(agg, m, dinv, b, p)


def _gn_cols(st, gw, gb, gm, eps=1e-5):
    mean = st[0:1] / N
    ex2 = st[1:2] / N
    var = ex2 - mean * mean * gm * (2.0 - gm)
    scale = gw / jnp.sqrt(var + eps)
    shift = gb - gm * mean * scale
    return scale, shift


def _gn_act_t_body(a_ref, st_ref, dinv_ref, gw_ref, gb_ref, gm_ref, p_ref,
                   h_ref, *t_refs, prelu_here):
    scale, shift = _gn_cols(st_ref[...], gw_ref[0], gb_ref[0], gm_ref[0])
    h = a_ref[...] * scale + shift
    if prelu_here:
        h = jnp.where(h >= 0, h, p_ref[0] * h)
    h_ref[...] = h
    t = dinv_ref[...] * h
    for n, t_ref in enumerate(t_refs):
        t_ref[...] = t[:, n * 128:(n + 1) * 128]


def _gn_act_t(a, st, dinv, gw, gb, gm, p, prelu_here=True):
    C = a.shape[1]
    NT = C // 128
    body = functools.partial(_gn_act_t_body, prelu_here=prelu_here)
    outs = pl.pallas_call(
        body,
        grid=(NBLK,),
        in_specs=[
            pl.BlockSpec((BLK, C), lambda i: (i, 0)),
            pl.BlockSpec((8, C), lambda i: (0, 0)),
            pl.BlockSpec((BLK, 1), lambda i: (i, 0)),
            pl.BlockSpec((1, C), lambda i: (0, 0)),
            pl.BlockSpec((1, C), lambda i: (0, 0)),
            pl.BlockSpec((1, C), lambda i: (0, 0)),
            pl.BlockSpec((1, C), lambda i: (0, 0)),
        ],
        out_specs=[pl.BlockSpec((BLK, C), lambda i: (i, 0))]
        + [pl.BlockSpec((BLK, 128), lambda i: (i, 0))] * NT,
        out_shape=[jax.ShapeDtypeStruct((N, C), _f32)]
        + [jax.ShapeDtypeStruct((N, 128), _f32)] * NT,
    )(a, st, dinv, gw, gb, gm, p)
    return outs[0], list(outs[1:])


def _gn_next_body(a_ref, st_ref, dinv_ref, gw_ref, gb_ref, gm_ref,
                  W_ref, m_ref, t_ref):
    scale, shift = _gn_cols(st_ref[...], gw_ref[0], gb_ref[0], gm_ref[0])
    h = a_ref[...] * scale + shift
    m = jnp.dot(h, W_ref[...], preferred_element_type=_f32)
    m_ref[...] = m
    t_ref[...] = dinv_ref[...] * m


def _gn_next(a, st, dinv, gw, gb, gm, W):
    Cin, Cout = W.shape
    return pl.pallas_call(
        _gn_next_body,
        grid=(NBLK,),
        in_specs=[
            pl.BlockSpec((BLK, Cin), lambda i: (i, 0)),
            pl.BlockSpec((8, Cin), lambda i: (0, 0)),
            pl.BlockSpec((BLK, 1), lambda i: (i, 0)),
            pl.BlockSpec((1, Cin), lambda i: (0, 0)),
            pl.BlockSpec((1, Cin), lambda i: (0, 0)),
            pl.BlockSpec((1, Cin), lambda i: (0, 0)),
            pl.BlockSpec((Cin, Cout), lambda i: (0, 0)),
        ],
        out_specs=[
            pl.BlockSpec((BLK, Cout), lambda i: (i, 0)),
            pl.BlockSpec((BLK, Cout), lambda i: (i, 0)),
        ],
        out_shape=[
            jax.ShapeDtypeStruct((N, Cout), _f32),
            jax.ShapeDtypeStruct((N, Cout), _f32),
        ],
    )(a, st, dinv, gw, gb, gm, W)


def _pool_body(a_ref, st_ref, gw_ref, gb_ref, gm_ref, batch_ref,
               l12W_ref, l12b_ref, l3W_ref, l3b_ref, p3_ref, out_ref,
               ssum, scnt, smax):
    i = pl.program_id(0)
    scale, shift = _gn_cols(st_ref[...], gw_ref[0], gb_ref[0], gm_ref[0])
    h = a_ref[...] * scale + shift

    @pl.when(i == 0)
    def _():
        ssum[...] = jnp.zeros_like(ssum)
        scnt[...] = jnp.zeros_like(scnt)
        smax[...] = jnp.full_like(smax, -jnp.inf)

    b = batch_ref[...]
    valid = _row_mask(BLK)
    gmin = b[0, 0]
    glast = jnp.where(i == NBLK - 1, b[LAST_VALID - 1, 0], b[BLK - 1, 0])
    gmax = jnp.clip(glast, gmin, NG - 1)

    def g_body(g, _):
        msk = (b == g) & valid
        hm = jnp.where(msk, h, 0.0)
        ssum[pl.ds(g, 1), :] = ssum[pl.ds(g, 1), :] + jnp.sum(
            hm, axis=0, keepdims=True)
        scnt[pl.ds(g, 1), :] = scnt[pl.ds(g, 1), :] + jnp.sum(
            msk.astype(_f32), axis=0, keepdims=True)
        hx = jnp.where(msk, h, -jnp.inf)
        smax[pl.ds(g, 1), :] = jnp.maximum(
            smax[pl.ds(g, 1), :], jnp.max(hx, axis=0, keepdims=True))
        return 0

    lax.fori_loop(gmin, gmax + 1, g_body, 0)

    @pl.when(i == NBLK - 1)
    def _():
        cnt = jnp.maximum(scnt[:, 0:1], 1.0)
        mean = ssum[...] / cnt
        z = jnp.concatenate([mean, smax[...]], axis=1)
        z = jnp.dot(z, l12W_ref[...], preferred_element_type=_f32) \
            + l12b_ref[0]
        z = jnp.where(z >= 0, z, p3_ref[0] * z)
        z = jnp.dot(z, l3W_ref[...], preferred_element_type=_f32) \
            + l3b_ref[0]
        out_ref[...] = z


def _pool(a4, st4, g4w, g4b, g4m, batch2d, l12W, l12b, l3W, l3b, p3):
    C = 64
    return pl.pallas_call(
        _pool_body,
        grid=(NBLK,),
        in_specs=[
            pl.BlockSpec((BLK, C), lambda i: (i, 0)),
            pl.BlockSpec((8, C), lambda i: (0, 0)),
            pl.BlockSpec((1, C), lambda i: (0, 0)),
            pl.BlockSpec((1, C), lambda i: (0, 0)),
            pl.BlockSpec((1, C), lambda i: (0, 0)),
            pl.BlockSpec((BLK, 1), lambda i: (i, 0)),
            pl.BlockSpec((2 * C, 32), lambda i: (0, 0)),
            pl.BlockSpec((1, 32), lambda i: (0, 0)),
            pl.BlockSpec((32, 1), lambda i: (0, 0)),
            pl.BlockSpec((1, 1), lambda i: (0, 0)),
            pl.BlockSpec((1, 32), lambda i: (0, 0)),
        ],
        out_specs=pl.BlockSpec((NG, 1), lambda i: (0, 0)),
        out_shape=jax.ShapeDtypeStruct((NG, 1), _f32),
        scratch_shapes=[
            pltpu.VMEM((NG, C), _f32),
            pltpu.VMEM((NG, C), _f32),
            pltpu.VMEM((NG, C), _f32),
        ],
    )(a4, st4, g4w, g4b, g4m, batch2d, l12W, l12b, l3W, l3b, p3)


def kernel(x, edge_index, edge_attr, batch, W1, b1, W2, b2, W3, b3, W4, b4,
           g1w, g1b, g1m, g2w, g2b, g2m, g3w, g3b, g3m, g4w, g4b, g4m,
           p0, p1, p2, p3, l12W, l12b, l3W, l3b):
    src = edge_index[0]
    dst = edge_index[1]
    npad_e = E_PAD - E
    src_pad = jnp.concatenate([src, jnp.zeros((npad_e,), _i32)])
    dst_pad = jnp.concatenate([dst, jnp.full((npad_e,), GARBAGE, _i32)])
    ew_pad = jnp.concatenate([edge_attr, jnp.zeros((npad_e,), _f32)])

    row2 = lambda v: v.reshape(1, -1)

    degpart2 = _deg_call(dst_pad, ew_pad).reshape(2 * 32, NPAD)
    dinv_a, dinv_b, t1 = _tc_a(degpart2, x)
    da, db = dinv_a[:N], dinv_b[:N]

    agg1 = _agg_call([t1], src_pad, dst_pad, ew_pad, False)[0][:N, :32]
    h1p, st1 = _conv_stats([agg1], x, da, W1, row2(b1), row2(p0),
                           prelu_here=False)
    h1, t2s = _gn_act_t(h1p, st1, db, row2(g1w), row2(g1b), row2(g1m),
                        row2(p0), prelu_here=True)

    agg2 = [a[:N] for a in _agg_call(t2s, src_pad, dst_pad, ew_pad, True)]
    a2, st2 = _conv_stats(agg2, h1, db, W2, row2(b2), row2(p0),
                          prelu_here=True)

    m3, t3 = _gn_next(a2, st2, db, row2(g2w), row2(g2b), row2(g2m), W3)
    agg3 = _agg_call([t3], src_pad, dst_pad, ew_pad, True)[0][:N]
    a3, st3 = _postagg_stats(agg3, m3, db, row2(b3), row2(p1))

    h3, t4s = _gn_act_t(a3, st3, db, row2(g3w), row2(g3b), row2(g3m),
                        row2(p1), prelu_here=False)
    agg4 = _agg_call(t4s, src_pad, dst_pad, ew_pad, True)[0][:N]
    a4, st4 = _conv_stats([agg4], h3, db, W4, row2(b4), row2(p2),
                          prelu_here=True)

    out = _pool(a4, st4, row2(g4w), row2(g4b), row2(g4m),
                batch.reshape(N, 1), l12W, row2(l12b), l3W,
                l3b.reshape(1, 1), row2(p3))
    return out.reshape(NG)


# speedup vs baseline: 6.8439x; 1.1018x over previous
import functools

import jax
import jax.numpy as jnp
from jax import lax
from jax.experimental import pallas as pl
from jax.experimental.pallas import tpu as pltpu
from jax.experimental.pallas import tpu_sc as plsc

N = 50000
E = 800000
NG = 64

NPAD = 50176
HALF = NPAD // 2
E_PAD = 802816
EP = E_PAD // 32
EPT = E_PAD // 16
CHK = 3136
NCHK = EP // CHK
NCHK_AGG = EPT // CHK
G = 128
STG = 6400
FLUSH_AT = 3072
GARBAGE = NPAD - 1

_f32 = jnp.float32
_i32 = jnp.int32


def _deg_body(dst_hbm, ew_hbm, out_hbm, dstbuf, ewbuf, hist_a, hist_b, sem):
    core = lax.axis_index("c")
    sub = lax.axis_index("s")
    tid = sub * 2 + core
    base = tid * EP

    def zero_body(i, _):
        z = jnp.zeros((16,), _f32)
        hist_a[pl.ds(i * 16, 16)] = z
        hist_b[pl.ds(i * 16, 16)] = z
        return 0

    lax.fori_loop(0, NPAD // 16, zero_body, 0)

    ones = jnp.ones((16,), _f32)

    def chunk_body(c, _):
        off = base + c * CHK
        pltpu.sync_copy(dst_hbm.at[pl.ds(off, CHK)], dstbuf)
        pltpu.sync_copy(ew_hbm.at[pl.ds(off, CHK)], ewbuf)

        def vec_body(v, _):
            d = dstbuf[pl.ds(v * 16, 16)]
            w = ewbuf[pl.ds(v * 16, 16)]
            plsc.addupdate_scatter(hist_a, [d], ones)
            plsc.addupdate_scatter(hist_b, [d], w)
            return 0

        lax.fori_loop(0, CHK // 16, vec_body, 0)
        return 0

    lax.fori_loop(0, NCHK, chunk_body, 0)

    pltpu.sync_copy(hist_a, out_hbm.at[0, tid])
    pltpu.sync_copy(hist_b, out_hbm.at[1, tid])


def _deg_call(dst_pad, ew_pad):
    mesh = plsc.VectorSubcoreMesh(core_axis_name="c", subcore_axis_name="s")
    return pl.kernel(
        _deg_body,
        out_type=jax.ShapeDtypeStruct((2, 32, NPAD), _f32),
        mesh=mesh,
        scratch_types=[
            pltpu.VMEM((CHK,), _i32),
            pltpu.VMEM((CHK,), _f32),
            pltpu.VMEM((NPAD,), _f32),
            pltpu.VMEM((NPAD,), _f32),
            pltpu.SemaphoreType.DMA,
        ],
        compiler_params=pltpu.CompilerParams(needs_layout_passes=False),
        name="sc_deg",
    )(dst_pad, ew_pad)


def _agg_body(*refs, NT, S, R, use_ew):
    t_hbms = refs[:NT]
    src_hbm, dst_hbm, ew_hbm = refs[NT:NT + 3]
    out_hbms = refs[NT + 3:2 * NT + 3]
    sc = refs[2 * NT + 3:]
    srcbuf, dstbuf, ewbuf, stg_src, stg_dst, stg_ew, idx2d = sc[:7]
    rowbufs = sc[7:7 + NT]
    accs = sc[7 + NT:7 + 2 * NT]
    sems_g = sc[7 + 2 * NT:9 + 2 * NT]
    sems_s = sc[9 + 2 * NT:11 + 2 * NT]

    core = lax.axis_index("c")
    sub = lax.axis_index("s")
    base = sub * EPT
    rpt = R // 16
    nvec_row = 128 // 16
    lanes = lax.iota(_i32, 16)

    def pad_stage(cnt):
        q = (cnt // 16) * 16
        m_pad = (q + lanes) >= cnt
        plsc.store_scatter(stg_src, [q + lanes], jnp.zeros((16,), _i32),
                           mask=m_pad)
        plsc.store_scatter(stg_dst, [q + lanes],
                           jnp.full((16,), R, _i32), mask=m_pad)
        if use_ew:
            plsc.store_scatter(stg_ew, [q + lanes], jnp.zeros((16,), _f32),
                               mask=m_pad)
        zi = jnp.zeros((16,), _i32)
        zf = jnp.zeros((16,), _f32)
        gr = jnp.full((16,), R, _i32)
        for j in range(1, 10):
            stg_src[pl.ds(q + 16 * j, 16)] = zi
            stg_dst[pl.ds(q + 16 * j, 16)] = gr
            if use_ew:
                stg_ew[pl.ds(q + 16 * j, 16)] = zf

    def issue_gather(j, par):
        for n in range(NT):
            pltpu.async_copy(t_hbms[n].at[stg_src.at[pl.ds(j * G, G)]],
                             rowbufs[n].at[par], sems_g[par])

    def wait_gather(par):
        for n in range(NT):
            pltpu.make_async_copy(t_hbms[n].at[stg_src.at[pl.ds(0, G)]],
                                  rowbufs[n].at[par], sems_g[par]).wait()

    def flush(cnt):
        pad_stage(cnt)
        nb = (cnt + (G - 1)) // G

        @pl.when(nb > 0)
        def _():
            issue_gather(0, 0)

        def outer_body(jj, _):
            for par in (0, 1):
                j = 2 * jj + par

                @pl.when(j < nb)
                def _():
                    @pl.when(j + 1 < nb)
                    def _():
                        issue_gather(j + 1, 1 - par)

                    wait_gather(par)
                    bb = j * G
                    if use_ew:
                        def scale_body(q, _):
                            wv = stg_ew[pl.ds(bb + q * 16, 16)]
                            for l in range(16):
                                s = wv[l]
                                e = q * 16 + l
                                for n in range(NT):
                                    for k in range(nvec_row):
                                        rowbufs[n][par, e,
                                                   pl.ds(k * 16, 16)] = (
                                            rowbufs[n][par, e,
                                                       pl.ds(k * 16, 16)]
                                            * s)
                            return 0
                        lax.fori_loop(0, G // 16, scale_body, 0)
                    for k in range(G // 16):
                        idx2d[par, pl.ds(k * 16, 16)] = (
                            stg_dst[pl.ds(bb + k * 16, 16)])
                    for n in range(NT):
                        pltpu.sync_copy(rowbufs[n].at[par],
                                        accs[n].at[idx2d.at[par]],
                                        add=True)
            return 0

        lax.fori_loop(0, (nb + 1) // 2, outer_body, 0)

    def pass_body(p, _):
        lo = core * HALF + p * R

        def zr_body(r, _):
            for n in range(NT):
                for k in range(nvec_row):
                    rowbufs[n][0, r, pl.ds(k * 16, 16)] = (
                        jnp.zeros((16,), _f32))
            return 0
        lax.fori_loop(0, G, zr_body, 0)

        nfull = rpt // G
        rem = rpt % G
        for n in range(NT):
            rb0 = rowbufs[n].at[0]
            for j in range(nfull):
                pltpu.sync_copy(rb0,
                                accs[n].at[pl.ds(sub * rpt + j * G, G)])
            if rem:
                pltpu.sync_copy(rb0.at[pl.ds(0, rem)],
                                accs[n].at[pl.ds(sub * rpt + nfull * G, rem)])
        plsc.subcore_barrier()

        def chunk_body(c, cnt):
            off = base + c * CHK
            pltpu.sync_copy(src_hbm.at[pl.ds(off, CHK)], srcbuf)
            pltpu.sync_copy(dst_hbm.at[pl.ds(off, CHK)], dstbuf)
            if use_ew:
                pltpu.sync_copy(ew_hbm.at[pl.ds(off, CHK)], ewbuf)

            def vec_body(v, cnt):
                d = dstbuf[pl.ds(v * 16, 16)]
                s = srcbuf[pl.ds(v * 16, 16)]
                m = (d >= lo) & (d < lo + R)
                mi = m.astype(_i32)
                cs = plsc.cumsum(mi)
                pos = cnt + cs - 1
                plsc.store_scatter(stg_src, [pos], s, mask=m)
                plsc.store_scatter(stg_dst, [pos], d - lo, mask=m)
                if use_ew:
                    w = ewbuf[pl.ds(v * 16, 16)]
                    plsc.store_scatter(stg_ew, [pos], w, mask=m)
                return cnt + jnp.sum(mi)

            cnt = lax.fori_loop(0, CHK // 16, vec_body, cnt)

            def do_flush(c2):
                flush(c2)
                return 0

            return lax.cond(cnt >= FLUSH_AT, do_flush, lambda c2: c2, cnt)

        cnt = lax.fori_loop(0, NCHK_AGG, chunk_body, 0)
        lax.cond(cnt > 0, do_flush_final := (lambda c2: (flush(c2), 0)[1]),
                 lambda c2: 0, cnt)
        plsc.subcore_barrier()

        for n in range(NT):
            pltpu.sync_copy(accs[n].at[pl.ds(sub * rpt, rpt)],
                            out_hbms[n].at[pl.ds(lo + sub * rpt, rpt)])
        plsc.subcore_barrier()
        return 0

    lax.fori_loop(0, S, pass_body, 0)


def _agg_call(tables, src_pad, dst_pad, ew_pad, use_ew):
    NT = len(tables)
    S = {1: 4, 2: 7}[NT]
    R = HALF // S
    mesh = plsc.VectorSubcoreMesh(core_axis_name="c", subcore_axis_name="s")
    body = functools.partial(_agg_body, NT=NT, S=S, R=R, use_ew=use_ew)
    out = pl.kernel(
        body,
        out_type=[jax.ShapeDtypeStruct((NPAD, 128), _f32)] * NT,
        mesh=mesh,
        scratch_types=(
            [
                pltpu.VMEM((CHK,), _i32),
                pltpu.VMEM((CHK,), _i32),
                pltpu.VMEM((CHK,), _f32),
                pltpu.VMEM((STG,), _i32),
                pltpu.VMEM((STG,), _i32),
                pltpu.VMEM((STG,), _f32),
                pltpu.VMEM((2, G), _i32),
            ]
            + [pltpu.VMEM((2, G, 128), _f32)] * NT
            + [pltpu.VMEM_SHARED((R + 16, 128), _f32)] * NT
            + [pltpu.SemaphoreType.DMA] * 4
        ),
        compiler_params=pltpu.CompilerParams(needs_layout_passes=False),
        name=f"sc_agg_nt{NT}",
    )(*tables, src_pad, dst_pad, ew_pad)
    return list(out) if isinstance(out, (list, tuple)) else [out]


BLK = 1024
NBLK = NPAD // BLK
LAST_VALID = N - (NBLK - 1) * BLK


def _row_mask(nrows):
    i = pl.program_id(0)
    rows = jax.lax.broadcasted_iota(_i32, (nrows, 1), 0) + i * BLK
    return rows < N


def _tc_a_body(degp_ref, x_ref, da_ref, db_ref, t1_ref):
    part = degp_ref[...]
    deg_a = jnp.sum(part[:32], axis=0) + 1.0
    deg_b = jnp.sum(part[32:], axis=0) + 1.0
    da = jax.lax.rsqrt(deg_a)[:, None]
    db = jax.lax.rsqrt(deg_b)[:, None]
    da_ref[...] = da
    db_ref[...] = db
    t1 = da * x_ref[...]
    t1_ref[...] = jnp.concatenate(
        [t1, jnp.zeros((t1.shape[0], 96), _f32)], axis=1)


def _tc_a(degpart2, x):
    return pl.pallas_call(
        _tc_a_body,
        grid=(NBLK,),
        in_specs=[
            pl.BlockSpec((64, BLK), lambda i: (0, i)),
            pl.BlockSpec((BLK, 32), lambda i: (i, 0)),
        ],
        out_specs=[
            pl.BlockSpec((BLK, 1), lambda i: (i, 0)),
            pl.BlockSpec((BLK, 1), lambda i: (i, 0)),
            pl.BlockSpec((BLK, 128), lambda i: (i, 0)),
        ],
        out_shape=[
            jax.ShapeDtypeStruct((NPAD, 1), _f32),
            jax.ShapeDtypeStruct((NPAD, 1), _f32),
            jax.ShapeDtypeStruct((N, 128), _f32),
        ],
    )(degpart2, x)


def _stats_accum(i, h, st_ref):
    mask = _row_mask(h.shape[0])
    hm = jnp.where(mask, h, 0.0)
    s1 = jnp.sum(hm, axis=0, keepdims=True)
    s2 = jnp.sum(hm * hm, axis=0, keepdims=True)
    st = jnp.concatenate([s1, s2, jnp.zeros((6, h.shape[1]), _f32)], axis=0)

    @pl.when(i == 0)
    def _():
        st_ref[...] = st

    @pl.when(i > 0)
    def _():
        st_ref[...] = st_ref[...] + st


def _conv_stats_body(*refs, prelu_here, nagg):
    agg_refs = refs[:nagg]
    hprev_ref, dinv_ref, W_ref, b_ref, p_ref, out_ref, st_ref = refs[nagg:]
    i = pl.program_id(0)
    dv = dinv_ref[...]
    agg = jnp.concatenate([r[...] for r in agg_refs], axis=1) \
        if nagg > 1 else agg_refs[0][...]
    u = dv * agg + (dv * dv) * hprev_ref[...]
    h = jnp.dot(u, W_ref[...], preferred_element_type=_f32) + b_ref[0]
    if prelu_here:
        h = jnp.where(h >= 0, h, p_ref[0] * h)
    out_ref[...] = h
    _stats_accum(i, h, st_ref)


def _conv_stats(aggs, hprev, dinv, W, b, p, prelu_here):
    Cin, Cout = W.shape
    Ca = aggs[0].shape[1]
    body = functools.partial(_conv_stats_body, prelu_here=prelu_here,
                             nagg=len(aggs))
    return pl.pallas_call(
        body,
        grid=(NBLK,),
        in_specs=[pl.BlockSpec((BLK, Ca), lambda i: (i, 0))] * len(aggs)
        + [
            pl.BlockSpec((BLK, Cin), lambda i: (i, 0)),
            pl.BlockSpec((BLK, 1), lambda i: (i, 0)),
            pl.BlockSpec((Cin, Cout), lambda i: (0, 0)),
            pl.BlockSpec((1, Cout), lambda i: (0, 0)),
            pl.BlockSpec((1, Cout), lambda i: (0, 0)),
        ],
        out_specs=[
            pl.BlockSpec((BLK, Cout), lambda i: (i, 0)),
            pl.BlockSpec((8, Cout), lambda i: (0, 0)),
        ],
        out_shape=[
            jax.ShapeDtypeStruct((N, Cout), _f32),
            jax.ShapeDtypeStruct((8, Cout), _f32),
        ],
    )(*aggs, hprev, dinv, W, b, p)


def _postagg_stats_body(agg_ref, m_ref, dinv_ref, b_ref, p_ref,
                        out_ref, st_ref):
    i = pl.program_id(0)
    dv = dinv_ref[...]
    h = dv * agg_ref[...] + (dv * dv) * m_ref[...] + b_ref[0]
    h = jnp.where(h >= 0, h, p_ref[0] * h)
    out_ref[...] = h
    _stats_accum(i, h, st_ref)


def _postagg_stats(agg, m, dinv, b, p):
    Cout = m.shape[1]
    return pl.pallas_call(
        _postagg_stats_body,
        grid=(NBLK,),
        in_specs=[
            pl.BlockSpec((BLK, Cout), lambda i: (i, 0)),
            pl.BlockSpec((BLK, Cout), lambda i: (i, 0)),
            pl.BlockSpec((BLK, 1), lambda i: (i, 0)),
            pl.BlockSpec((1, Cout), lambda i: (0, 0)),
            pl.BlockSpec((1, Cout), lambda i: (0, 0)),
        ],
        out_specs=[
            pl.BlockSpec((BLK, Cout), lambda i: (i, 0)),
            pl.BlockSpec((8, Cout), lambda i: (0, 0)),
        ],
        out_shape=[
            jax.ShapeDtypeStruct((N, Cout), _f32),
            jax.ShapeDtypeStruct((8, Cout), _f32),
        ],
    )(agg, m, dinv, b, p)


def _gn_cols(st, gw, gb, gm, eps=1e-5):
    mean = st[0:1] / N
    ex2 = st[1:2] / N
    var = ex2 - mean * mean * gm * (2.0 - gm)
    scale = gw / jnp.sqrt(var + eps)
    shift = gb - gm * mean * scale
    return scale, shift


def _gn_act_t_body(a_ref, st_ref, dinv_ref, gw_ref, gb_ref, gm_ref, p_ref,
                   h_ref, *t_refs, prelu_here):
    scale, shift = _gn_cols(st_ref[...], gw_ref[0], gb_ref[0], gm_ref[0])
    h = a_ref[...] * scale + shift
    if prelu_here:
        h = jnp.where(h >= 0, h, p_ref[0] * h)
    h_ref[...] = h
    t = dinv_ref[...] * h
    for n, t_ref in enumerate(t_refs):
        t_ref[...] = t[:, n * 128:(n + 1) * 128]


def _gn_act_t(a, st, dinv, gw, gb, gm, p, prelu_here=True):
    C = a.shape[1]
    NT = C // 128
    body = functools.partial(_gn_act_t_body, prelu_here=prelu_here)
    outs = pl.pallas_call(
        body,
        grid=(NBLK,),
        in_specs=[
            pl.BlockSpec((BLK, C), lambda i: (i, 0)),
            pl.BlockSpec((8, C), lambda i: (0, 0)),
            pl.BlockSpec((BLK, 1), lambda i: (i, 0)),
            pl.BlockSpec((1, C), lambda i: (0, 0)),
            pl.BlockSpec((1, C), lambda i: (0, 0)),
            pl.BlockSpec((1, C), lambda i: (0, 0)),
            pl.BlockSpec((1, C), lambda i: (0, 0)),
        ],
        out_specs=[pl.BlockSpec((BLK, C), lambda i: (i, 0))]
        + [pl.BlockSpec((BLK, 128), lambda i: (i, 0))] * NT,
        out_shape=[jax.ShapeDtypeStruct((N, C), _f32)]
        + [jax.ShapeDtypeStruct((N, 128), _f32)] * NT,
    )(a, st, dinv, gw, gb, gm, p)
    return outs[0], list(outs[1:])


def _gn_next_body(a_ref, st_ref, dinv_ref, gw_ref, gb_ref, gm_ref,
                  W_ref, m_ref, t_ref):
    scale, shift = _gn_cols(st_ref[...], gw_ref[0], gb_ref[0], gm_ref[0])
    h = a_ref[...] * scale + shift
    m = jnp.dot(h, W_ref[...], preferred_element_type=_f32)
    m_ref[...] = m
    t_ref[...] = dinv_ref[...] * m


def _gn_next(a, st, dinv, gw, gb, gm, W):
    Cin, Cout = W.shape
    return pl.pallas_call(
        _gn_next_body,
        grid=(NBLK,),
        in_specs=[
            pl.BlockSpec((BLK, Cin), lambda i: (i, 0)),
            pl.BlockSpec((8, Cin), lambda i: (0, 0)),
            pl.BlockSpec((BLK, 1), lambda i: (i, 0)),
            pl.BlockSpec((1, Cin), lambda i: (0, 0)),
            pl.BlockSpec((1, Cin), lambda i: (0, 0)),
            pl.BlockSpec((1, Cin), lambda i: (0, 0)),
            pl.BlockSpec((Cin, Cout), lambda i: (0, 0)),
        ],
        out_specs=[
            pl.BlockSpec((BLK, Cout), lambda i: (i, 0)),
            pl.BlockSpec((BLK, Cout), lambda i: (i, 0)),
        ],
        out_shape=[
            jax.ShapeDtypeStruct((N, Cout), _f32),
            jax.ShapeDtypeStruct((N, Cout), _f32),
        ],
    )(a, st, dinv, gw, gb, gm, W)


def _pool_body(a_ref, st_ref, gw_ref, gb_ref, gm_ref, batch_ref,
               l12W_ref, l12b_ref, l3W_ref, l3b_ref, p3_ref, out_ref,
               ssum, scnt, smax):
    i = pl.program_id(0)
    scale, shift = _gn_cols(st_ref[...], gw_ref[0], gb_ref[0], gm_ref[0])
    h = a_ref[...] * scale + shift

    @pl.when(i == 0)
    def _():
        ssum[...] = jnp.zeros_like(ssum)
        scnt[...] = jnp.zeros_like(scnt)
        smax[...] = jnp.full_like(smax, -jnp.inf)

    b = batch_ref[...]
    valid = _row_mask(BLK)
    gmin = b[0, 0]
    glast = jnp.where(i == NBLK - 1, b[LAST_VALID - 1, 0], b[BLK - 1, 0])
    gmax = jnp.clip(glast, gmin, NG - 1)

    def g_body(g, _):
        msk = (b == g) & valid
        hm = jnp.where(msk, h, 0.0)
        ssum[pl.ds(g, 1), :] = ssum[pl.ds(g, 1), :] + jnp.sum(
            hm, axis=0, keepdims=True)
        scnt[pl.ds(g, 1), :] = scnt[pl.ds(g, 1), :] + jnp.sum(
            msk.astype(_f32), axis=0, keepdims=True)
        hx = jnp.where(msk, h, -jnp.inf)
        smax[pl.ds(g, 1), :] = jnp.maximum(
            smax[pl.ds(g, 1), :], jnp.max(hx, axis=0, keepdims=True))
        return 0

    lax.fori_loop(gmin, gmax + 1, g_body, 0)

    @pl.when(i == NBLK - 1)
    def _():
        cnt = jnp.maximum(scnt[:, 0:1], 1.0)
        mean = ssum[...] / cnt
        z = jnp.concatenate([mean, smax[...]], axis=1)
        z = jnp.dot(z, l12W_ref[...], preferred_element_type=_f32) \
            + l12b_ref[0]
        z = jnp.where(z >= 0, z, p3_ref[0] * z)
        z = jnp.dot(z, l3W_ref[...], preferred_element_type=_f32) \
            + l3b_ref[0]
        out_ref[...] = z


def _pool(a4, st4, g4w, g4b, g4m, batch2d, l12W, l12b, l3W, l3b, p3):
    C = 64
    return pl.pallas_call(
        _pool_body,
        grid=(NBLK,),
        in_specs=[
            pl.BlockSpec((BLK, C), lambda i: (i, 0)),
            pl.BlockSpec((8, C), lambda i: (0, 0)),
            pl.BlockSpec((1, C), lambda i: (0, 0)),
            pl.BlockSpec((1, C), lambda i: (0, 0)),
            pl.BlockSpec((1, C), lambda i: (0, 0)),
            pl.BlockSpec((BLK, 1), lambda i: (i, 0)),
            pl.BlockSpec((2 * C, 32), lambda i: (0, 0)),
            pl.BlockSpec((1, 32), lambda i: (0, 0)),
            pl.BlockSpec((32, 1), lambda i: (0, 0)),
            pl.BlockSpec((1, 1), lambda i: (0, 0)),
            pl.BlockSpec((1, 32), lambda i: (0, 0)),
        ],
        out_specs=pl.BlockSpec((NG, 1), lambda i: (0, 0)),
        out_shape=jax.ShapeDtypeStruct((NG, 1), _f32),
        scratch_shapes=[
            pltpu.VMEM((NG, C), _f32),
            pltpu.VMEM((NG, C), _f32),
            pltpu.VMEM((NG, C), _f32),
        ],
    )(a4, st4, g4w, g4b, g4m, batch2d, l12W, l12b, l3W, l3b, p3)


def kernel(x, edge_index, edge_attr, batch, W1, b1, W2, b2, W3, b3, W4, b4,
           g1w, g1b, g1m, g2w, g2b, g2m, g3w, g3b, g3m, g4w, g4b, g4m,
           p0, p1, p2, p3, l12W, l12b, l3W, l3b):
    src = edge_index[0]
    dst = edge_index[1]
    npad_e = E_PAD - E
    src_pad = jnp.concatenate([src, jnp.zeros((npad_e,), _i32)])
    dst_pad = jnp.concatenate([dst, jnp.full((npad_e,), GARBAGE, _i32)])
    ew_pad = jnp.concatenate([edge_attr, jnp.zeros((npad_e,), _f32)])

    row2 = lambda v: v.reshape(1, -1)

    degpart2 = _deg_call(dst_pad, ew_pad).reshape(2 * 32, NPAD)
    dinv_a, dinv_b, t1 = _tc_a(degpart2, x)
    da, db = dinv_a[:N], dinv_b[:N]

    agg1 = _agg_call([t1], src_pad, dst_pad, ew_pad, False)[0][:N, :32]
    h1p, st1 = _conv_stats([agg1], x, da, W1, row2(b1), row2(p0),
                           prelu_here=False)
    h1, t2s = _gn_act_t(h1p, st1, db, row2(g1w), row2(g1b), row2(g1m),
                        row2(p0), prelu_here=True)

    agg2 = [_agg_call([t], src_pad, dst_pad, ew_pad, True)[0][:N]
            for t in t2s]
    a2, st2 = _conv_stats(agg2, h1, db, W2, row2(b2), row2(p0),
                          prelu_here=True)

    m3, t3 = _gn_next(a2, st2, db, row2(g2w), row2(g2b), row2(g2m), W3)
    agg3 = _agg_call([t3], src_pad, dst_pad, ew_pad, True)[0][:N]
    a3, st3 = _postagg_stats(agg3, m3, db, row2(b3), row2(p1))

    h3, t4s = _gn_act_t(a3, st3, db, row2(g3w), row2(g3b), row2(g3m),
                        row2(p1), prelu_here=False)
    agg4 = _agg_call(t4s, src_pad, dst_pad, ew_pad, True)[0][:N]
    a4, st4 = _conv_stats([agg4], h3, db, W4, row2(b4), row2(p2),
                          prelu_here=True)

    out = _pool(a4, st4, row2(g4w), row2(g4b), row2(g4m),
                batch.reshape(N, 1), l12W, row2(l12b), l3W,
                l3b.reshape(1, 1), row2(p3))
    return out.reshape(NG)


# speedup vs baseline: 7.1816x; 1.0493x over previous
import functools

import jax
import jax.numpy as jnp
from jax import lax
from jax.experimental import pallas as pl
from jax.experimental.pallas import tpu as pltpu
from jax.experimental.pallas import tpu_sc as plsc

N = 50000
E = 800000
NG = 64

NPAD = 50176
HALF = NPAD // 2
E_PAD = 802816
EP = E_PAD // 32
EPT = E_PAD // 16
CHK = 3136
NCHK = EP // CHK
NCHK_AGG = EPT // CHK
G = 128
STG = 6400
FLUSH_AT = 3072
GARBAGE = NPAD - 1

_f32 = jnp.float32
_i32 = jnp.int32


def _deg_body(dst_hbm, ew_hbm, out_hbm, dstbuf, ewbuf, hist_a, hist_b, sem):
    core = lax.axis_index("c")
    sub = lax.axis_index("s")
    tid = sub * 2 + core
    base = tid * EP

    def zero_body(i, _):
        z = jnp.zeros((16,), _f32)
        hist_a[pl.ds(i * 16, 16)] = z
        hist_b[pl.ds(i * 16, 16)] = z
        return 0

    lax.fori_loop(0, NPAD // 16, zero_body, 0)

    ones = jnp.ones((16,), _f32)

    def chunk_body(c, _):
        off = base + c * CHK
        pltpu.sync_copy(dst_hbm.at[pl.ds(off, CHK)], dstbuf)
        pltpu.sync_copy(ew_hbm.at[pl.ds(off, CHK)], ewbuf)

        def vec_body(v, _):
            d = dstbuf[pl.ds(v * 16, 16)]
            w = ewbuf[pl.ds(v * 16, 16)]
            plsc.addupdate_scatter(hist_a, [d], ones)
            plsc.addupdate_scatter(hist_b, [d], w)
            return 0

        lax.fori_loop(0, CHK // 16, vec_body, 0)
        return 0

    lax.fori_loop(0, NCHK, chunk_body, 0)

    pltpu.sync_copy(hist_a, out_hbm.at[0, tid])
    pltpu.sync_copy(hist_b, out_hbm.at[1, tid])


def _deg_call(dst_pad, ew_pad):
    mesh = plsc.VectorSubcoreMesh(core_axis_name="c", subcore_axis_name="s")
    return pl.kernel(
        _deg_body,
        out_type=jax.ShapeDtypeStruct((2, 32, NPAD), _f32),
        mesh=mesh,
        scratch_types=[
            pltpu.VMEM((CHK,), _i32),
            pltpu.VMEM((CHK,), _f32),
            pltpu.VMEM((NPAD,), _f32),
            pltpu.VMEM((NPAD,), _f32),
            pltpu.SemaphoreType.DMA,
        ],
        compiler_params=pltpu.CompilerParams(needs_layout_passes=False),
        name="sc_deg",
    )(dst_pad, ew_pad)


def _agg_body(*refs, NT, S, R, use_ew):
    t_hbms = refs[:NT]
    src_hbm, dst_hbm, ew_hbm = refs[NT:NT + 3]
    out_hbms = refs[NT + 3:2 * NT + 3]
    sc = refs[2 * NT + 3:]
    srcbuf, dstbuf, ewbuf, stg_src, stg_dst, stg_ew, idx2d = sc[:7]
    rowbufs = sc[7:7 + NT]
    accs = sc[7 + NT:7 + 2 * NT]
    sems_g = sc[7 + 2 * NT:9 + 2 * NT]
    sems_s = sc[9 + 2 * NT:11 + 2 * NT]

    core = lax.axis_index("c")
    sub = lax.axis_index("s")
    base = sub * EPT
    rpt = R // 16
    nvec_row = 128 // 16
    lanes = lax.iota(_i32, 16)

    def pad_stage(cnt):
        q = (cnt // 16) * 16
        m_pad = (q + lanes) >= cnt
        plsc.store_scatter(stg_src, [q + lanes], jnp.zeros((16,), _i32),
                           mask=m_pad)
        plsc.store_scatter(stg_dst, [q + lanes],
                           jnp.full((16,), R, _i32), mask=m_pad)
        if use_ew:
            plsc.store_scatter(stg_ew, [q + lanes], jnp.zeros((16,), _f32),
                               mask=m_pad)
        zi = jnp.zeros((16,), _i32)
        zf = jnp.zeros((16,), _f32)
        gr = jnp.full((16,), R, _i32)
        for j in range(1, 10):
            stg_src[pl.ds(q + 16 * j, 16)] = zi
            stg_dst[pl.ds(q + 16 * j, 16)] = gr
            if use_ew:
                stg_ew[pl.ds(q + 16 * j, 16)] = zf

    def issue_gather(j, par):
        for n in range(NT):
            pltpu.async_copy(t_hbms[n].at[stg_src.at[pl.ds(j * G, G)]],
                             rowbufs[n].at[par], sems_g[par])

    def wait_gather(par):
        for n in range(NT):
            pltpu.make_async_copy(t_hbms[n].at[stg_src.at[pl.ds(0, G)]],
                                  rowbufs[n].at[par], sems_g[par]).wait()

    def flush(cnt):
        pad_stage(cnt)
        nb = (cnt + (G - 1)) // G

        @pl.when(nb > 0)
        def _():
            issue_gather(0, 0)

        def outer_body(jj, _):
            for par in (0, 1):
                j = 2 * jj + par

                @pl.when(j < nb)
                def _():
                    @pl.when(j + 1 < nb)
                    def _():
                        issue_gather(j + 1, 1 - par)

                    wait_gather(par)
                    bb = j * G
                    if use_ew:
                        def scale_body(q, _):
                            wv = stg_ew[pl.ds(bb + q * 16, 16)]
                            for l in range(16):
                                s = wv[l]
                                e = q * 16 + l
                                for n in range(NT):
                                    for k in range(nvec_row):
                                        rowbufs[n][par, e,
                                                   pl.ds(k * 16, 16)] = (
                                            rowbufs[n][par, e,
                                                       pl.ds(k * 16, 16)]
                                            * s)
                            return 0
                        lax.fori_loop(0, G // 16, scale_body, 0)
                    for k in range(G // 16):
                        idx2d[par, pl.ds(k * 16, 16)] = (
                            stg_dst[pl.ds(bb + k * 16, 16)])
                    for n in range(NT):
                        pltpu.sync_copy(rowbufs[n].at[par],
                                        accs[n].at[idx2d.at[par]],
                                        add=True)
            return 0

        lax.fori_loop(0, (nb + 1) // 2, outer_body, 0)

    def pass_body(p, _):
        lo = core * HALF + p * R

        def zr_body(r, _):
            for n in range(NT):
                for k in range(nvec_row):
                    rowbufs[n][0, r, pl.ds(k * 16, 16)] = (
                        jnp.zeros((16,), _f32))
            return 0
        lax.fori_loop(0, G, zr_body, 0)

        nfull = rpt // G
        rem = rpt % G
        for n in range(NT):
            rb0 = rowbufs[n].at[0]
            for j in range(nfull):
                pltpu.sync_copy(rb0,
                                accs[n].at[pl.ds(sub * rpt + j * G, G)])
            if rem:
                pltpu.sync_copy(rb0.at[pl.ds(0, rem)],
                                accs[n].at[pl.ds(sub * rpt + nfull * G, rem)])
        plsc.subcore_barrier()

        def chunk_body(c, cnt):
            off = base + c * CHK
            loads = [
                pltpu.async_copy(src_hbm.at[pl.ds(off, CHK)], srcbuf,
                                 sems_s[0]),
                pltpu.async_copy(dst_hbm.at[pl.ds(off, CHK)], dstbuf,
                                 sems_s[1]),
            ]
            if use_ew:
                loads.append(pltpu.async_copy(ew_hbm.at[pl.ds(off, CHK)],
                                              ewbuf, sems_s[0]))
            for ld in loads:
                ld.wait()

            def vec_body(v, cnt):
                d = dstbuf[pl.ds(v * 16, 16)]
                s = srcbuf[pl.ds(v * 16, 16)]
                m = (d >= lo) & (d < lo + R)
                mi = m.astype(_i32)
                cs = plsc.cumsum(mi)
                pos = cnt + cs - 1
                plsc.store_scatter(stg_src, [pos], s, mask=m)
                plsc.store_scatter(stg_dst, [pos], d - lo, mask=m)
                if use_ew:
                    w = ewbuf[pl.ds(v * 16, 16)]
                    plsc.store_scatter(stg_ew, [pos], w, mask=m)
                return cnt + jnp.sum(mi)

            cnt = lax.fori_loop(0, CHK // 16, vec_body, cnt)

            def do_flush(c2):
                flush(c2)
                return 0

            return lax.cond(cnt >= FLUSH_AT, do_flush, lambda c2: c2, cnt)

        cnt = lax.fori_loop(0, NCHK_AGG, chunk_body, 0)
        lax.cond(cnt > 0, do_flush_final := (lambda c2: (flush(c2), 0)[1]),
                 lambda c2: 0, cnt)
        plsc.subcore_barrier()

        for n in range(NT):
            pltpu.sync_copy(accs[n].at[pl.ds(sub * rpt, rpt)],
                            out_hbms[n].at[pl.ds(lo + sub * rpt, rpt)])
        plsc.subcore_barrier()
        return 0

    lax.fori_loop(0, S, pass_body, 0)


def _agg_call(tables, src_pad, dst_pad, ew_pad, use_ew):
    NT = len(tables)
    S = {1: 4, 2: 7}[NT]
    R = HALF // S
    mesh = plsc.VectorSubcoreMesh(core_axis_name="c", subcore_axis_name="s")
    body = functools.partial(_agg_body, NT=NT, S=S, R=R, use_ew=use_ew)
    out = pl.kernel(
        body,
        out_type=[jax.ShapeDtypeStruct((NPAD, 128), _f32)] * NT,
        mesh=mesh,
        scratch_types=(
            [
                pltpu.VMEM((CHK,), _i32),
                pltpu.VMEM((CHK,), _i32),
                pltpu.VMEM((CHK,), _f32),
                pltpu.VMEM((STG,), _i32),
                pltpu.VMEM((STG,), _i32),
                pltpu.VMEM((STG,), _f32),
                pltpu.VMEM((2, G), _i32),
            ]
            + [pltpu.VMEM((2, G, 128), _f32)] * NT
            + [pltpu.VMEM_SHARED((R + 16, 128), _f32)] * NT
            + [pltpu.SemaphoreType.DMA] * 4
        ),
        compiler_params=pltpu.CompilerParams(needs_layout_passes=False),
        name=f"sc_agg_nt{NT}",
    )(*tables, src_pad, dst_pad, ew_pad)
    return list(out) if isinstance(out, (list, tuple)) else [out]


BLK = 1024
NBLK = NPAD // BLK
LAST_VALID = N - (NBLK - 1) * BLK


def _row_mask(nrows):
    i = pl.program_id(0)
    rows = jax.lax.broadcasted_iota(_i32, (nrows, 1), 0) + i * BLK
    return rows < N


def _tc_a_body(degp_ref, x_ref, da_ref, db_ref, t1_ref):
    part = degp_ref[...]
    deg_a = jnp.sum(part[:32], axis=0) + 1.0
    deg_b = jnp.sum(part[32:], axis=0) + 1.0
    da = jax.lax.rsqrt(deg_a)[:, None]
    db = jax.lax.rsqrt(deg_b)[:, None]
    da_ref[...] = da
    db_ref[...] = db
    t1 = da * x_ref[...]
    t1_ref[...] = jnp.concatenate(
        [t1, jnp.zeros((t1.shape[0], 96), _f32)], axis=1)


def _tc_a(degpart2, x):
    return pl.pallas_call(
        _tc_a_body,
        grid=(NBLK,),
        in_specs=[
            pl.BlockSpec((64, BLK), lambda i: (0, i)),
            pl.BlockSpec((BLK, 32), lambda i: (i, 0)),
        ],
        out_specs=[
            pl.BlockSpec((BLK, 1), lambda i: (i, 0)),
            pl.BlockSpec((BLK, 1), lambda i: (i, 0)),
            pl.BlockSpec((BLK, 128), lambda i: (i, 0)),
        ],
        out_shape=[
            jax.ShapeDtypeStruct((NPAD, 1), _f32),
            jax.ShapeDtypeStruct((NPAD, 1), _f32),
            jax.ShapeDtypeStruct((N, 128), _f32),
        ],
    )(degpart2, x)


def _stats_accum(i, h, st_ref):
    mask = _row_mask(h.shape[0])
    hm = jnp.where(mask, h, 0.0)
    s1 = jnp.sum(hm, axis=0, keepdims=True)
    s2 = jnp.sum(hm * hm, axis=0, keepdims=True)
    st = jnp.concatenate([s1, s2, jnp.zeros((6, h.shape[1]), _f32)], axis=0)

    @pl.when(i == 0)
    def _():
        st_ref[...] = st

    @pl.when(i > 0)
    def _():
        st_ref[...] = st_ref[...] + st


def _conv_stats_body(*refs, prelu_here, nagg):
    agg_refs = refs[:nagg]
    hprev_ref, dinv_ref, W_ref, b_ref, p_ref, out_ref, st_ref = refs[nagg:]
    i = pl.program_id(0)
    dv = dinv_ref[...]
    agg = jnp.concatenate([r[...] for r in agg_refs], axis=1) \
        if nagg > 1 else agg_refs[0][...]
    u = dv * agg + (dv * dv) * hprev_ref[...]
    h = jnp.dot(u, W_ref[...], preferred_element_type=_f32) + b_ref[0]
    if prelu_here:
        h = jnp.where(h >= 0, h, p_ref[0] * h)
    out_ref[...] = h
    _stats_accum(i, h, st_ref)


def _conv_stats(aggs, hprev, dinv, W, b, p, prelu_here):
    Cin, Cout = W.shape
    Ca = aggs[0].shape[1]
    body = functools.partial(_conv_stats_body, prelu_here=prelu_here,
                             nagg=len(aggs))
    return pl.pallas_call(
        body,
        grid=(NBLK,),
        in_specs=[pl.BlockSpec((BLK, Ca), lambda i: (i, 0))] * len(aggs)
        + [
            pl.BlockSpec((BLK, Cin), lambda i: (i, 0)),
            pl.BlockSpec((BLK, 1), lambda i: (i, 0)),
            pl.BlockSpec((Cin, Cout), lambda i: (0, 0)),
            pl.BlockSpec((1, Cout), lambda i: (0, 0)),
            pl.BlockSpec((1, Cout), lambda i: (0, 0)),
        ],
        out_specs=[
            pl.BlockSpec((BLK, Cout), lambda i: (i, 0)),
            pl.BlockSpec((8, Cout), lambda i: (0, 0)),
        ],
        out_shape=[
            jax.ShapeDtypeStruct((N, Cout), _f32),
            jax.ShapeDtypeStruct((8, Cout), _f32),
        ],
    )(*aggs, hprev, dinv, W, b, p)


def _postagg_stats_body(agg_ref, m_ref, dinv_ref, b_ref, p_ref,
                        out_ref, st_ref):
    i = pl.program_id(0)
    dv = dinv_ref[...]
    h = dv * agg_ref[...] + (dv * dv) * m_ref[...] + b_ref[0]
    h = jnp.where(h >= 0, h, p_ref[0] * h)
    out_ref[...] = h
    _stats_accum(i, h, st_ref)


def _postagg_stats(agg, m, dinv, b, p):
    Cout = m.shape[1]
    return pl.pallas_call(
        _postagg_stats_body,
        grid=(NBLK,),
        in_specs=[
            pl.BlockSpec((BLK, Cout), lambda i: (i, 0)),
            pl.BlockSpec((BLK, Cout), lambda i: (i, 0)),
            pl.BlockSpec((BLK, 1), lambda i: (i, 0)),
            pl.BlockSpec((1, Cout), lambda i: (0, 0)),
            pl.BlockSpec((1, Cout), lambda i: (0, 0)),
        ],
        out_specs=[
            pl.BlockSpec((BLK, Cout), lambda i: (i, 0)),
            pl.BlockSpec((8, Cout), lambda i: (0, 0)),
        ],
        out_shape=[
            jax.ShapeDtypeStruct((N, Cout), _f32),
            jax.ShapeDtypeStruct((8, Cout), _f32),
        ],
    )(agg, m, dinv, b, p)


def _gn_cols(st, gw, gb, gm, eps=1e-5):
    mean = st[0:1] / N
    ex2 = st[1:2] / N
    var = ex2 - mean * mean * gm * (2.0 - gm)
    scale = gw / jnp.sqrt(var + eps)
    shift = gb - gm * mean * scale
    return scale, shift


def _gn_act_t_body(a_ref, st_ref, dinv_ref, gw_ref, gb_ref, gm_ref, p_ref,
                   h_ref, *t_refs, prelu_here):
    scale, shift = _gn_cols(st_ref[...], gw_ref[0], gb_ref[0], gm_ref[0])
    h = a_ref[...] * scale + shift
    if prelu_here:
        h = jnp.where(h >= 0, h, p_ref[0] * h)
    h_ref[...] = h
    t = dinv_ref[...] * h
    for n, t_ref in enumerate(t_refs):
        t_ref[...] = t[:, n * 128:(n + 1) * 128]


def _gn_act_t(a, st, dinv, gw, gb, gm, p, prelu_here=True):
    C = a.shape[1]
    NT = C // 128
    body = functools.partial(_gn_act_t_body, prelu_here=prelu_here)
    outs = pl.pallas_call(
        body,
        grid=(NBLK,),
        in_specs=[
            pl.BlockSpec((BLK, C), lambda i: (i, 0)),
            pl.BlockSpec((8, C), lambda i: (0, 0)),
            pl.BlockSpec((BLK, 1), lambda i: (i, 0)),
            pl.BlockSpec((1, C), lambda i: (0, 0)),
            pl.BlockSpec((1, C), lambda i: (0, 0)),
            pl.BlockSpec((1, C), lambda i: (0, 0)),
            pl.BlockSpec((1, C), lambda i: (0, 0)),
        ],
        out_specs=[pl.BlockSpec((BLK, C), lambda i: (i, 0))]
        + [pl.BlockSpec((BLK, 128), lambda i: (i, 0))] * NT,
        out_shape=[jax.ShapeDtypeStruct((N, C), _f32)]
        + [jax.ShapeDtypeStruct((N, 128), _f32)] * NT,
    )(a, st, dinv, gw, gb, gm, p)
    return outs[0], list(outs[1:])


def _gn_next_body(a_ref, st_ref, dinv_ref, gw_ref, gb_ref, gm_ref,
                  W_ref, m_ref, t_ref):
    scale, shift = _gn_cols(st_ref[...], gw_ref[0], gb_ref[0], gm_ref[0])
    h = a_ref[...] * scale + shift
    m = jnp.dot(h, W_ref[...], preferred_element_type=_f32)
    m_ref[...] = m
    t_ref[...] = dinv_ref[...] * m


def _gn_next(a, st, dinv, gw, gb, gm, W):
    Cin, Cout = W.shape
    return pl.pallas_call(
        _gn_next_body,
        grid=(NBLK,),
        in_specs=[
            pl.BlockSpec((BLK, Cin), lambda i: (i, 0)),
            pl.BlockSpec((8, Cin), lambda i: (0, 0)),
            pl.BlockSpec((BLK, 1), lambda i: (i, 0)),
            pl.BlockSpec((1, Cin), lambda i: (0, 0)),
            pl.BlockSpec((1, Cin), lambda i: (0, 0)),
            pl.BlockSpec((1, Cin), lambda i: (0, 0)),
            pl.BlockSpec((Cin, Cout), lambda i: (0, 0)),
        ],
        out_specs=[
            pl.BlockSpec((BLK, Cout), lambda i: (i, 0)),
            pl.BlockSpec((BLK, Cout), lambda i: (i, 0)),
        ],
        out_shape=[
            jax.ShapeDtypeStruct((N, Cout), _f32),
            jax.ShapeDtypeStruct((N, Cout), _f32),
        ],
    )(a, st, dinv, gw, gb, gm, W)


def _pool_body(a_ref, st_ref, gw_ref, gb_ref, gm_ref, batch_ref,
               l12W_ref, l12b_ref, l3W_ref, l3b_ref, p3_ref, out_ref,
               ssum, scnt, smax):
    i = pl.program_id(0)
    scale, shift = _gn_cols(st_ref[...], gw_ref[0], gb_ref[0], gm_ref[0])
    h = a_ref[...] * scale + shift

    @pl.when(i == 0)
    def _():
        ssum[...] = jnp.zeros_like(ssum)
        scnt[...] = jnp.zeros_like(scnt)
        smax[...] = jnp.full_like(smax, -jnp.inf)

    b = batch_ref[...]
    valid = _row_mask(BLK)
    gmin = b[0, 0]
    glast = jnp.where(i == NBLK - 1, b[LAST_VALID - 1, 0], b[BLK - 1, 0])
    gmax = jnp.clip(glast, gmin, NG - 1)

    def g_body(g, _):
        msk = (b == g) & valid
        hm = jnp.where(msk, h, 0.0)
        ssum[pl.ds(g, 1), :] = ssum[pl.ds(g, 1), :] + jnp.sum(
            hm, axis=0, keepdims=True)
        scnt[pl.ds(g, 1), :] = scnt[pl.ds(g, 1), :] + jnp.sum(
            msk.astype(_f32), axis=0, keepdims=True)
        hx = jnp.where(msk, h, -jnp.inf)
        smax[pl.ds(g, 1), :] = jnp.maximum(
            smax[pl.ds(g, 1), :], jnp.max(hx, axis=0, keepdims=True))
        return 0

    lax.fori_loop(gmin, gmax + 1, g_body, 0)

    @pl.when(i == NBLK - 1)
    def _():
        cnt = jnp.maximum(scnt[:, 0:1], 1.0)
        mean = ssum[...] / cnt
        z = jnp.concatenate([mean, smax[...]], axis=1)
        z = jnp.dot(z, l12W_ref[...], preferred_element_type=_f32) \
            + l12b_ref[0]
        z = jnp.where(z >= 0, z, p3_ref[0] * z)
        z = jnp.dot(z, l3W_ref[...], preferred_element_type=_f32) \
            + l3b_ref[0]
        out_ref[...] = z


def _pool(a4, st4, g4w, g4b, g4m, batch2d, l12W, l12b, l3W, l3b, p3):
    C = 64
    return pl.pallas_call(
        _pool_body,
        grid=(NBLK,),
        in_specs=[
            pl.BlockSpec((BLK, C), lambda i: (i, 0)),
            pl.BlockSpec((8, C), lambda i: (0, 0)),
            pl.BlockSpec((1, C), lambda i: (0, 0)),
            pl.BlockSpec((1, C), lambda i: (0, 0)),
            pl.BlockSpec((1, C), lambda i: (0, 0)),
            pl.BlockSpec((BLK, 1), lambda i: (i, 0)),
            pl.BlockSpec((2 * C, 32), lambda i: (0, 0)),
            pl.BlockSpec((1, 32), lambda i: (0, 0)),
            pl.BlockSpec((32, 1), lambda i: (0, 0)),
            pl.BlockSpec((1, 1), lambda i: (0, 0)),
            pl.BlockSpec((1, 32), lambda i: (0, 0)),
        ],
        out_specs=pl.BlockSpec((NG, 1), lambda i: (0, 0)),
        out_shape=jax.ShapeDtypeStruct((NG, 1), _f32),
        scratch_shapes=[
            pltpu.VMEM((NG, C), _f32),
            pltpu.VMEM((NG, C), _f32),
            pltpu.VMEM((NG, C), _f32),
        ],
    )(a4, st4, g4w, g4b, g4m, batch2d, l12W, l12b, l3W, l3b, p3)


def kernel(x, edge_index, edge_attr, batch, W1, b1, W2, b2, W3, b3, W4, b4,
           g1w, g1b, g1m, g2w, g2b, g2m, g3w, g3b, g3m, g4w, g4b, g4m,
           p0, p1, p2, p3, l12W, l12b, l3W, l3b):
    src = edge_index[0]
    dst = edge_index[1]
    npad_e = E_PAD - E
    src_pad = jnp.concatenate([src, jnp.zeros((npad_e,), _i32)])
    dst_pad = jnp.concatenate([dst, jnp.full((npad_e,), GARBAGE, _i32)])
    ew_pad = jnp.concatenate([edge_attr, jnp.zeros((npad_e,), _f32)])

    row2 = lambda v: v.reshape(1, -1)

    degpart2 = _deg_call(dst_pad, ew_pad).reshape(2 * 32, NPAD)
    dinv_a, dinv_b, t1 = _tc_a(degpart2, x)
    da, db = dinv_a[:N], dinv_b[:N]

    agg1 = _agg_call([t1], src_pad, dst_pad, ew_pad, False)[0][:N, :32]
    h1p, st1 = _conv_stats([agg1], x, da, W1, row2(b1), row2(p0),
                           prelu_here=False)
    h1, t2s = _gn_act_t(h1p, st1, db, row2(g1w), row2(g1b), row2(g1m),
                        row2(p0), prelu_here=True)

    agg2 = [_agg_call([t], src_pad, dst_pad, ew_pad, True)[0][:N]
            for t in t2s]
    a2, st2 = _conv_stats(agg2, h1, db, W2, row2(b2), row2(p0),
                          prelu_here=True)

    m3, t3 = _gn_next(a2, st2, db, row2(g2w), row2(g2b), row2(g2m), W3)
    agg3 = _agg_call([t3], src_pad, dst_pad, ew_pad, True)[0][:N]
    a3, st3 = _postagg_stats(agg3, m3, db, row2(b3), row2(p1))

    h3, t4s = _gn_act_t(a3, st3, db, row2(g3w), row2(g3b), row2(g3m),
                        row2(p1), prelu_here=False)
    agg4 = _agg_call(t4s, src_pad, dst_pad, ew_pad, True)[0][:N]
    a4, st4 = _conv_stats([agg4], h3, db, W4, row2(b4), row2(p2),
                          prelu_here=True)

    out = _pool(a4, st4, row2(g4w), row2(g4b), row2(g4m),
                batch.reshape(N, 1), l12W, row2(l12b), l3W,
                l3b.reshape(1, 1), row2(p3))
    return out.reshape(NG)
